# Initial kernel scaffold; baseline (speedup 1.0000x reference)
import jax
import jax.numpy as jnp
from jax.experimental import pallas as pl


def kernel(latents, weight):
    raise NotImplementedError("write your pallas kernel here")


# speedup vs baseline: 5.0724x; 5.0724x over previous
import functools

import jax
import jax.numpy as jnp
from jax import lax
from jax.experimental import pallas as pl
from jax.experimental.pallas import tpu as pltpu
from jax.experimental.pallas import tpu_sc as plsc


def _tc_body(z_ref, wt_ref, wsq_ref, inds_ref, loss_ref):
    b = pl.program_id(0)
    z = z_ref[0]
    wt = wt_ref[...]
    wsq = wsq_ref[...]

    d = lax.dot_general(
        z, wt, (((1,), (0,)), ((), ())),
        precision=lax.Precision.HIGHEST,
        preferred_element_type=jnp.float32,
    )
    score = wsq - 2.0 * d

    m = jnp.min(score, axis=1, keepdims=True)
    kiota = lax.broadcasted_iota(jnp.int32, score.shape, 1)
    inds = jnp.min(jnp.where(score <= m, kiota, score.shape[1]),
                   axis=1, keepdims=True)
    inds_ref[0] = inds

    dl = jnp.sum(jnp.where(kiota == inds, d, 0.0),
                 axis=1, keepdims=True)
    mx = jnp.max(dl, axis=0, keepdims=True)
    e = jnp.exp(dl - mx)
    s = jnp.sum(e, axis=0, keepdims=True)
    part = jnp.sum(e / s, axis=0, keepdims=True)

    @pl.when(b == 0)
    def _():
        loss_ref[...] = jnp.zeros_like(loss_ref)

    loss_ref[...] = loss_ref[...] + part


def _sc_gather(table, idx):
    V, D = table.shape
    (N,) = idx.shape
    info = plsc.get_sparse_core_info()
    nw = info.num_cores * info.num_subcores
    assert N % (8 * nw) == 0 and D % info.num_lanes == 0
    n_per_w = N // nw
    mesh = plsc.VectorSubcoreMesh(core_axis_name="c", subcore_axis_name="s")

    @functools.partial(
        pl.kernel, mesh=mesh,
        out_type=jax.ShapeDtypeStruct((N, D), jnp.float32),
        compiler_params=pltpu.CompilerParams(use_tc_tiling_on_sc=False),
        scratch_types=[
            pltpu.VMEM((n_per_w,), jnp.int32),
            pltpu.VMEM((n_per_w, D), jnp.float32),
            pltpu.SemaphoreType.DMA,
        ],
    )
    def k(table_hbm, idx_hbm, out_hbm, idx_v, rows_v, sem):
        wid = lax.axis_index("s") * info.num_cores + lax.axis_index("c")
        base = wid * n_per_w
        pltpu.sync_copy(idx_hbm.at[pl.ds(base, n_per_w)], idx_v)
        pltpu.async_copy(table_hbm.at[idx_v], rows_v, sem).wait()
        pltpu.sync_copy(rows_v, out_hbm.at[pl.ds(base, n_per_w)])

    return k(table, idx)


def kernel(latents, weight):
    B, T, D = latents.shape
    K = weight.shape[0]
    wt = weight.T
    wsq = jnp.sum(weight * weight, axis=1)[None, :]

    inds3, loss = pl.pallas_call(
        _tc_body,
        grid=(B,),
        in_specs=[
            pl.BlockSpec((1, T, D), lambda b: (b, 0, 0)),
            pl.BlockSpec((D, K), lambda b: (0, 0)),
            pl.BlockSpec((1, K), lambda b: (0, 0)),
        ],
        out_specs=[
            pl.BlockSpec((1, T, 1), lambda b: (b, 0, 0)),
            pl.BlockSpec((1, 1), lambda b: (0, 0)),
        ],
        out_shape=[
            jax.ShapeDtypeStruct((B, T, 1), jnp.int32),
            jax.ShapeDtypeStruct((1, 1), jnp.float32),
        ],
    )(latents, wt, wsq)

    q = _sc_gather(weight, inds3.reshape(B * T))
    return (q.reshape(B, T, D), loss[0, 0])


# speedup vs baseline: 5.3460x; 1.0539x over previous
import functools

import jax
import jax.numpy as jnp
from jax import lax
from jax.experimental import pallas as pl
from jax.experimental.pallas import tpu as pltpu
from jax.experimental.pallas import tpu_sc as plsc


def _tc_body(nbatch, z_ref, wt_ref, inds_ref, loss_ref):
    z = z_ref[...]
    wt = wt_ref[...]
    wsq = jnp.sum(wt * wt, axis=0, keepdims=True)

    d = lax.dot_general(
        z, wt, (((1,), (0,)), ((), ())),
        precision=lax.Precision.HIGHEST,
        preferred_element_type=jnp.float32,
    )
    score = wsq - 2.0 * d

    m = jnp.min(score, axis=1, keepdims=True)
    kiota = lax.broadcasted_iota(jnp.int32, score.shape, 1)
    inds = jnp.min(jnp.where(score <= m, kiota, score.shape[1]),
                   axis=1, keepdims=True)
    inds_ref[...] = inds

    dl = jnp.sum(jnp.where(kiota == inds, d, 0.0),
                 axis=1, keepdims=True)
    t_per_b = dl.shape[0] // nbatch
    acc = jnp.zeros((1, 1), jnp.float32)
    for b in range(nbatch):
        seg = dl[b * t_per_b:(b + 1) * t_per_b]
        mx = jnp.max(seg, axis=0, keepdims=True)
        e = jnp.exp(seg - mx)
        s = jnp.sum(e, axis=0, keepdims=True)
        acc = acc + jnp.sum(e / s, axis=0, keepdims=True)
    loss_ref[...] = acc


def _sc_gather(table, idx):
    V, D = table.shape
    (N,) = idx.shape
    info = plsc.get_sparse_core_info()
    nw = info.num_cores * info.num_subcores
    assert N % (8 * nw) == 0 and D % info.num_lanes == 0
    n_per_w = N // nw
    mesh = plsc.VectorSubcoreMesh(core_axis_name="c", subcore_axis_name="s")

    @functools.partial(
        pl.kernel, mesh=mesh,
        out_type=jax.ShapeDtypeStruct((N, D), jnp.float32),
        compiler_params=pltpu.CompilerParams(use_tc_tiling_on_sc=False),
        scratch_types=[
            pltpu.VMEM((n_per_w,), jnp.int32),
            pltpu.VMEM((n_per_w, D), jnp.float32),
            pltpu.SemaphoreType.DMA,
        ],
    )
    def k(table_hbm, idx_hbm, out_hbm, idx_v, rows_v, sem):
        wid = lax.axis_index("s") * info.num_cores + lax.axis_index("c")
        base = wid * n_per_w
        pltpu.sync_copy(idx_hbm.at[pl.ds(base, n_per_w)], idx_v)
        pltpu.async_copy(table_hbm.at[idx_v], rows_v, sem).wait()
        pltpu.sync_copy(rows_v, out_hbm.at[pl.ds(base, n_per_w)])

    return k(table, idx)


def kernel(latents, weight):
    B, T, D = latents.shape
    N = B * T

    inds2, loss = pl.pallas_call(
        functools.partial(_tc_body, B),
        out_shape=[
            jax.ShapeDtypeStruct((N, 1), jnp.int32),
            jax.ShapeDtypeStruct((1, 1), jnp.float32),
        ],
    )(latents.reshape(N, D), weight.T)

    q = _sc_gather(weight, inds2.reshape(N))
    return (q.reshape(B, T, D), loss[0, 0])
